# baseline (speedup 1.0000x reference)
import functools

import jax
import jax.numpy as jnp
from jax import lax
from jax.experimental import pallas as pl
from jax.experimental.pallas import tpu as pltpu
from jax.experimental.pallas import tpu_sc as plsc

B, S, D, E = 2, 2048, 1024, 8
T = B * S
EP = 128
M = 512
MM = 256
NB = T // M
NBM = T // MM
WMAX = NBM + E - 1
XCOL = D + 128

NW = 32
TPW = T // NW
CS = 32
NCH = TPW // CS


def _gate_body(x_ref, wg_ref, tri_ref, gate_ref, rank_ref, psum_ref, cnt_ref,
               xsc_ref, run_ref):
    b = pl.program_id(0)

    @pl.when(b == 0)
    def _():
        run_ref[...] = jnp.zeros_like(run_ref)

    xb = x_ref[...]
    wg = wg_ref[...]
    logits = jnp.dot(xb, wg, preferred_element_type=jnp.float32)
    lane = lax.broadcasted_iota(jnp.int32, (M, EP), 1)
    valid = lane < E
    neg = jnp.full_like(logits, -jnp.inf)
    logit_m = jnp.where(valid, logits, neg)
    mx = jnp.max(logit_m, axis=-1, keepdims=True)
    ex = jnp.where(valid, jnp.exp(logit_m - mx), 0.0)
    den = jnp.sum(ex, axis=-1, keepdims=True)
    probs = ex / den
    gate = jnp.argmax(logit_m, axis=-1).astype(jnp.int32)
    selp = jnp.max(probs, axis=-1)

    onehot = jnp.where(lane == gate[:, None], 1.0, 0.0)
    cum_excl = jnp.dot(tri_ref[...], onehot, preferred_element_type=jnp.float32)
    local_rank = jnp.sum(cum_excl * onehot, axis=1)
    carry = jnp.sum(run_ref[...] * onehot, axis=1)
    rank = (local_rank + carry).astype(jnp.int32)

    gate_ref[0, 0, :] = gate
    rank_ref[0, 0, :] = rank
    psum_ref[0, 0, :] = jnp.sum(probs, axis=0)
    cnt_ref[0, 0, :] = jnp.sum(onehot, axis=0)
    run_ref[...] = run_ref[...] + jnp.sum(onehot, axis=0, keepdims=True)

    xsc_ref[:, :D] = xb * selp[:, None]
    lane2 = lax.broadcasted_iota(jnp.int32, (M, XCOL - D), 1)
    xsc_ref[:, D:] = jnp.where(lane2 == 0, selp[:, None], 0.0)


def _pos_chunks(gate_hbm, rank_hbm, off_hbm, g_v, r_v, o_v, pos_v, base):
    pltpu.sync_copy(gate_hbm.at[pl.ds(base, TPW)], g_v)
    pltpu.sync_copy(rank_hbm.at[pl.ds(base, TPW)], r_v)
    pltpu.sync_copy(off_hbm, o_v)
    for c in range(TPW // 16):
        g16 = g_v[pl.ds(c * 16, 16)]
        off16 = plsc.load_gather(o_v, [g16])
        j, k = divmod(c * 16, CS)
        pos_v[j, pl.ds(k, 16)] = off16 + r_v[pl.ds(c * 16, 16)]


def _dispatch_body(gate_hbm, rank_hbm, off_hbm, xsc_hbm, xs_hbm,
                   g_v, r_v, o_v, pos_v, rows_v, sem):
    wid = lax.axis_index("s") * 2 + lax.axis_index("c")
    base = wid * TPW
    _pos_chunks(gate_hbm, rank_hbm, off_hbm, g_v, r_v, o_v, pos_v, base)
    for j in range(NCH):
        pltpu.sync_copy(xsc_hbm.at[pl.ds(base + j * CS, CS)], rows_v)
        pltpu.async_copy(rows_v, xs_hbm.at[pos_v.at[j]], sem).wait()


def _combine_body(gate_hbm, rank_hbm, off_hbm, ys_hbm, out_hbm,
                  g_v, r_v, o_v, pos_v, rows_v, sem):
    wid = lax.axis_index("s") * 2 + lax.axis_index("c")
    base = wid * TPW
    _pos_chunks(gate_hbm, rank_hbm, off_hbm, g_v, r_v, o_v, pos_v, base)
    for j in range(NCH):
        pltpu.async_copy(ys_hbm.at[pos_v.at[j]], rows_v, sem).wait()
        pltpu.sync_copy(rows_v, out_hbm.at[pl.ds(base + j * CS, CS)])


@functools.cache
def _sc_kernels():
    mesh = plsc.VectorSubcoreMesh(core_axis_name="c", subcore_axis_name="s")
    params = pltpu.CompilerParams(needs_layout_passes=False)
    dispatch = pl.kernel(
        _dispatch_body, mesh=mesh, compiler_params=params,
        out_type=jax.ShapeDtypeStruct((T, XCOL), jnp.float32),
        scratch_types=[
            pltpu.VMEM((TPW,), jnp.int32),
            pltpu.VMEM((TPW,), jnp.int32),
            pltpu.VMEM((16,), jnp.int32),
            pltpu.VMEM((NCH, CS), jnp.int32),
            pltpu.VMEM((CS, XCOL), jnp.float32),
            pltpu.SemaphoreType.DMA,
        ],
    )
    combine = pl.kernel(
        _combine_body, mesh=mesh, compiler_params=params,
        out_type=jax.ShapeDtypeStruct((T, D), jnp.float32),
        scratch_types=[
            pltpu.VMEM((TPW,), jnp.int32),
            pltpu.VMEM((TPW,), jnp.int32),
            pltpu.VMEM((16,), jnp.int32),
            pltpu.VMEM((NCH, CS), jnp.int32),
            pltpu.VMEM((CS, D), jnp.float32),
            pltpu.SemaphoreType.DMA,
        ],
    )
    return dispatch, combine


def _moe_body(blk_s, eid_s, rs_s, re_s, xs_ref, w_ref, b_ref, ys_ref):
    del eid_s
    w = pl.program_id(0)
    blk = blk_s[w]
    prev_blk = blk_s[jnp.maximum(w - 1, 0)]
    first = jnp.logical_or(w == 0, blk != prev_blk)
    xb = xs_ref[...]
    y = jnp.dot(xb[:, :D], w_ref[0], preferred_element_type=jnp.float32)
    y = y + xb[:, D:D + 1] * b_ref[0, 0, :][None, :]
    jg = blk * MM + lax.broadcasted_iota(jnp.int32, (MM, 1), 0)
    mask = jnp.logical_and(jg >= rs_s[w], jg < re_s[w])
    contrib = jnp.where(mask, y, 0.0)
    ys_ref[...] = jnp.where(first, contrib, ys_ref[...] + contrib)


def kernel(x, attention_mask, W_gate, W_experts, b_experts):
    del attention_mask
    xf = x.reshape(T, D)
    wg_pad = jnp.zeros((D, EP), jnp.float32).at[:, :E].set(W_gate)

    gate3, rank3, psum3, cnt3, xsc = pl.pallas_call(
        _gate_body,
        grid=(NB,),
        in_specs=[
            pl.BlockSpec((M, D), lambda b: (b, 0)),
            pl.BlockSpec((D, EP), lambda b: (0, 0)),
            pl.BlockSpec((M, M), lambda b: (0, 0)),
        ],
        out_specs=[
            pl.BlockSpec((1, 1, M), lambda b: (b, 0, 0)),
            pl.BlockSpec((1, 1, M), lambda b: (b, 0, 0)),
            pl.BlockSpec((1, 1, EP), lambda b: (b, 0, 0)),
            pl.BlockSpec((1, 1, EP), lambda b: (b, 0, 0)),
            pl.BlockSpec((M, XCOL), lambda b: (b, 0)),
        ],
        out_shape=[
            jax.ShapeDtypeStruct((NB, 1, M), jnp.int32),
            jax.ShapeDtypeStruct((NB, 1, M), jnp.int32),
            jax.ShapeDtypeStruct((NB, 1, EP), jnp.float32),
            jax.ShapeDtypeStruct((NB, 1, EP), jnp.float32),
            jax.ShapeDtypeStruct((T, XCOL), jnp.float32),
        ],
        scratch_shapes=[pltpu.VMEM((1, EP), jnp.float32)],
    )(xf, wg_pad, jnp.tril(jnp.ones((M, M), jnp.float32), -1))

    gate = gate3.reshape(T)
    rank = rank3.reshape(T)
    counts_f = jnp.sum(cnt3[:, 0, :E], axis=0)
    counts = counts_f.astype(jnp.int32)
    off = jnp.concatenate([jnp.zeros((1,), jnp.int32), jnp.cumsum(counts)])
    off_pad = jnp.concatenate([off, jnp.full((16 - E - 1,), T, jnp.int32)])

    _dispatch, _combine = _sc_kernels()
    xs = _dispatch(gate, rank, off_pad, xsc)

    bb = jnp.arange(NBM, dtype=jnp.int32)[:, None]
    ee = jnp.arange(E, dtype=jnp.int32)[None, :]
    seg_s = jnp.maximum(off[ee], bb * MM)
    seg_e = jnp.minimum(off[ee + 1], (bb + 1) * MM)
    active = (seg_e > seg_s).reshape(-1)
    cpos = jnp.cumsum(active.astype(jnp.int32)) - 1
    slot = jnp.where(active, cpos, WMAX)
    bb_f = jnp.broadcast_to(bb, (NBM, E)).reshape(-1)
    ee_f = jnp.broadcast_to(ee, (NBM, E)).reshape(-1)

    def scat(init, vals):
        return jnp.full((WMAX + 1,), init, jnp.int32).at[slot].set(vals)[:WMAX]

    blk_l = scat(NBM - 1, bb_f)
    eid_l = scat(0, ee_f)
    rs_l = scat(0, seg_s.reshape(-1))
    re_l = scat(0, seg_e.reshape(-1))

    grid_spec = pltpu.PrefetchScalarGridSpec(
        num_scalar_prefetch=4,
        grid=(WMAX,),
        in_specs=[
            pl.BlockSpec((MM, XCOL), lambda w, blk, eid, rs, re: (blk[w], 0)),
            pl.BlockSpec((1, D, D), lambda w, blk, eid, rs, re: (eid[w], 0, 0)),
            pl.BlockSpec((1, 1, D), lambda w, blk, eid, rs, re: (eid[w], 0, 0)),
        ],
        out_specs=pl.BlockSpec((MM, D), lambda w, blk, eid, rs, re: (blk[w], 0)),
    )
    ys = pl.pallas_call(
        _moe_body,
        grid_spec=grid_spec,
        out_shape=jax.ShapeDtypeStruct((T, D), jnp.float32),
    )(blk_l, eid_l, rs_l, re_l, xs, W_experts, b_experts.reshape(E, 1, D))

    out = _combine(gate, rank, off_pad, ys)

    psum = jnp.sum(psum3[:, 0, :E], axis=0)
    P = psum / jnp.float32(T)
    f = counts_f / jnp.sum(counts_f)
    balance_loss = jnp.float32(E) * jnp.sum(P * f)
    return out.reshape(B, S, D), balance_loss, counts


if __name__ == "__main__":
    pass


# speedup vs baseline: 1.0070x; 1.0070x over previous
import functools

import jax
import jax.numpy as jnp
from jax import lax
from jax.experimental import pallas as pl
from jax.experimental.pallas import tpu as pltpu
from jax.experimental.pallas import tpu_sc as plsc

B, S, D, E = 2, 2048, 1024, 8
T = B * S
EP = 128
M = 512
MM = 512
NB = T // M
NBM = T // MM
WMAX = NBM + E - 1
XCOL = D + 128

NW = 32
TPW = T // NW
CS = 32
NCH = TPW // CS


def _gate_body(x_ref, wg_ref, tri_ref, gate_ref, rank_ref, psum_ref, cnt_ref,
               xsc_ref, run_ref):
    b = pl.program_id(0)

    @pl.when(b == 0)
    def _():
        run_ref[...] = jnp.zeros_like(run_ref)

    xb = x_ref[...]
    wg = wg_ref[...]
    logits = jnp.dot(xb, wg, preferred_element_type=jnp.float32)
    lane = lax.broadcasted_iota(jnp.int32, (M, EP), 1)
    valid = lane < E
    neg = jnp.full_like(logits, -jnp.inf)
    logit_m = jnp.where(valid, logits, neg)
    mx = jnp.max(logit_m, axis=-1, keepdims=True)
    ex = jnp.where(valid, jnp.exp(logit_m - mx), 0.0)
    den = jnp.sum(ex, axis=-1, keepdims=True)
    probs = ex / den
    gate = jnp.argmax(logit_m, axis=-1).astype(jnp.int32)
    selp = jnp.max(probs, axis=-1)

    onehot = jnp.where(lane == gate[:, None], 1.0, 0.0)
    cum_excl = jnp.dot(tri_ref[...], onehot, preferred_element_type=jnp.float32)
    local_rank = jnp.sum(cum_excl * onehot, axis=1)
    carry = jnp.sum(run_ref[...] * onehot, axis=1)
    rank = (local_rank + carry).astype(jnp.int32)

    gate_ref[0, 0, :] = gate
    rank_ref[0, 0, :] = rank
    psum_ref[0, 0, :] = jnp.sum(probs, axis=0)
    cnt_ref[0, 0, :] = jnp.sum(onehot, axis=0)
    run_ref[...] = run_ref[...] + jnp.sum(onehot, axis=0, keepdims=True)

    xsc_ref[:, :D] = xb * selp[:, None]
    lane2 = lax.broadcasted_iota(jnp.int32, (M, XCOL - D), 1)
    xsc_ref[:, D:] = jnp.where(lane2 == 0, selp[:, None], 0.0)


def _pos_chunks(gate_hbm, rank_hbm, off_hbm, g_v, r_v, o_v, pos_v, base):
    pltpu.sync_copy(gate_hbm.at[pl.ds(base, TPW)], g_v)
    pltpu.sync_copy(rank_hbm.at[pl.ds(base, TPW)], r_v)
    pltpu.sync_copy(off_hbm, o_v)
    for c in range(TPW // 16):
        g16 = g_v[pl.ds(c * 16, 16)]
        off16 = plsc.load_gather(o_v, [g16])
        j, k = divmod(c * 16, CS)
        pos_v[j, pl.ds(k, 16)] = off16 + r_v[pl.ds(c * 16, 16)]


def _dispatch_body(gate_hbm, rank_hbm, off_hbm, xsc_hbm, xs_hbm,
                   g_v, r_v, o_v, pos_v, rows_v, sem):
    wid = lax.axis_index("s") * 2 + lax.axis_index("c")
    base = wid * TPW
    _pos_chunks(gate_hbm, rank_hbm, off_hbm, g_v, r_v, o_v, pos_v, base)
    for j in range(NCH):
        pltpu.sync_copy(xsc_hbm.at[pl.ds(base + j * CS, CS)], rows_v)
        pltpu.async_copy(rows_v, xs_hbm.at[pos_v.at[j]], sem).wait()


def _combine_body(gate_hbm, rank_hbm, off_hbm, ys_hbm, out_hbm,
                  g_v, r_v, o_v, pos_v, rows_v, sem):
    wid = lax.axis_index("s") * 2 + lax.axis_index("c")
    base = wid * TPW
    _pos_chunks(gate_hbm, rank_hbm, off_hbm, g_v, r_v, o_v, pos_v, base)
    for j in range(NCH):
        pltpu.async_copy(ys_hbm.at[pos_v.at[j]], rows_v, sem).wait()
        pltpu.sync_copy(rows_v, out_hbm.at[pl.ds(base + j * CS, CS)])


@functools.cache
def _sc_kernels():
    mesh = plsc.VectorSubcoreMesh(core_axis_name="c", subcore_axis_name="s")
    params = pltpu.CompilerParams(needs_layout_passes=False)
    dispatch = pl.kernel(
        _dispatch_body, mesh=mesh, compiler_params=params,
        out_type=jax.ShapeDtypeStruct((T, XCOL), jnp.float32),
        scratch_types=[
            pltpu.VMEM((TPW,), jnp.int32),
            pltpu.VMEM((TPW,), jnp.int32),
            pltpu.VMEM((16,), jnp.int32),
            pltpu.VMEM((NCH, CS), jnp.int32),
            pltpu.VMEM((CS, XCOL), jnp.float32),
            pltpu.SemaphoreType.DMA,
        ],
    )
    combine = pl.kernel(
        _combine_body, mesh=mesh, compiler_params=params,
        out_type=jax.ShapeDtypeStruct((T, D), jnp.float32),
        scratch_types=[
            pltpu.VMEM((TPW,), jnp.int32),
            pltpu.VMEM((TPW,), jnp.int32),
            pltpu.VMEM((16,), jnp.int32),
            pltpu.VMEM((NCH, CS), jnp.int32),
            pltpu.VMEM((CS, D), jnp.float32),
            pltpu.SemaphoreType.DMA,
        ],
    )
    return dispatch, combine


def _moe_body(blk_s, eid_s, rs_s, re_s, xs_ref, w_ref, b_ref, ys_ref):
    del eid_s
    w = pl.program_id(0)
    blk = blk_s[w]
    prev_blk = blk_s[jnp.maximum(w - 1, 0)]
    first = jnp.logical_or(w == 0, blk != prev_blk)
    xb = xs_ref[...]
    y = jnp.dot(xb[:, :D], w_ref[0], preferred_element_type=jnp.float32)
    y = y + xb[:, D:D + 1] * b_ref[0, 0, :][None, :]
    jg = blk * MM + lax.broadcasted_iota(jnp.int32, (MM, 1), 0)
    mask = jnp.logical_and(jg >= rs_s[w], jg < re_s[w])
    contrib = jnp.where(mask, y, 0.0)
    ys_ref[...] = jnp.where(first, contrib, ys_ref[...] + contrib)


def kernel(x, attention_mask, W_gate, W_experts, b_experts):
    del attention_mask
    xf = x.reshape(T, D)
    wg_pad = jnp.zeros((D, EP), jnp.float32).at[:, :E].set(W_gate)

    gate3, rank3, psum3, cnt3, xsc = pl.pallas_call(
        _gate_body,
        grid=(NB,),
        in_specs=[
            pl.BlockSpec((M, D), lambda b: (b, 0)),
            pl.BlockSpec((D, EP), lambda b: (0, 0)),
            pl.BlockSpec((M, M), lambda b: (0, 0)),
        ],
        out_specs=[
            pl.BlockSpec((1, 1, M), lambda b: (b, 0, 0)),
            pl.BlockSpec((1, 1, M), lambda b: (b, 0, 0)),
            pl.BlockSpec((1, 1, EP), lambda b: (b, 0, 0)),
            pl.BlockSpec((1, 1, EP), lambda b: (b, 0, 0)),
            pl.BlockSpec((M, XCOL), lambda b: (b, 0)),
        ],
        out_shape=[
            jax.ShapeDtypeStruct((NB, 1, M), jnp.int32),
            jax.ShapeDtypeStruct((NB, 1, M), jnp.int32),
            jax.ShapeDtypeStruct((NB, 1, EP), jnp.float32),
            jax.ShapeDtypeStruct((NB, 1, EP), jnp.float32),
            jax.ShapeDtypeStruct((T, XCOL), jnp.float32),
        ],
        scratch_shapes=[pltpu.VMEM((1, EP), jnp.float32)],
    )(xf, wg_pad, jnp.tril(jnp.ones((M, M), jnp.float32), -1))

    gate = gate3.reshape(T)
    rank = rank3.reshape(T)
    counts_f = jnp.sum(cnt3[:, 0, :E], axis=0)
    counts = counts_f.astype(jnp.int32)
    off = jnp.concatenate([jnp.zeros((1,), jnp.int32), jnp.cumsum(counts)])
    off_pad = jnp.concatenate([off, jnp.full((16 - E - 1,), T, jnp.int32)])

    _dispatch, _combine = _sc_kernels()
    xs = _dispatch(gate, rank, off_pad, xsc)

    bb = jnp.arange(NBM, dtype=jnp.int32)[:, None]
    ee = jnp.arange(E, dtype=jnp.int32)[None, :]
    seg_s = jnp.maximum(off[ee], bb * MM)
    seg_e = jnp.minimum(off[ee + 1], (bb + 1) * MM)
    active = (seg_e > seg_s).reshape(-1)
    cpos = jnp.cumsum(active.astype(jnp.int32)) - 1
    slot = jnp.where(active, cpos, WMAX)
    bb_f = jnp.broadcast_to(bb, (NBM, E)).reshape(-1)
    ee_f = jnp.broadcast_to(ee, (NBM, E)).reshape(-1)

    def scat(init, vals):
        return jnp.full((WMAX + 1,), init, jnp.int32).at[slot].set(vals)[:WMAX]

    blk_l = scat(NBM - 1, bb_f)
    eid_l = scat(0, ee_f)
    rs_l = scat(0, seg_s.reshape(-1))
    re_l = scat(0, seg_e.reshape(-1))

    grid_spec = pltpu.PrefetchScalarGridSpec(
        num_scalar_prefetch=4,
        grid=(WMAX,),
        in_specs=[
            pl.BlockSpec((MM, XCOL), lambda w, blk, eid, rs, re: (blk[w], 0)),
            pl.BlockSpec((1, D, D), lambda w, blk, eid, rs, re: (eid[w], 0, 0)),
            pl.BlockSpec((1, 1, D), lambda w, blk, eid, rs, re: (eid[w], 0, 0)),
        ],
        out_specs=pl.BlockSpec((MM, D), lambda w, blk, eid, rs, re: (blk[w], 0)),
    )
    ys = pl.pallas_call(
        _moe_body,
        grid_spec=grid_spec,
        out_shape=jax.ShapeDtypeStruct((T, D), jnp.float32),
    )(blk_l, eid_l, rs_l, re_l, xs, W_experts, b_experts.reshape(E, 1, D))

    out = _combine(gate, rank, off_pad, ys)

    psum = jnp.sum(psum3[:, 0, :E], axis=0)
    P = psum / jnp.float32(T)
    f = counts_f / jnp.sum(counts_f)
    balance_loss = jnp.float32(E) * jnp.sum(P * f)
    return out.reshape(B, S, D), balance_loss, counts


if __name__ == "__main__":
    pass


# speedup vs baseline: 1.0481x; 1.0409x over previous
import functools

import jax
import jax.numpy as jnp
from jax import lax
from jax.experimental import pallas as pl
from jax.experimental.pallas import tpu as pltpu
from jax.experimental.pallas import tpu_sc as plsc

B, S, D, E = 2, 2048, 1024, 8
T = B * S
EP = 128
M = 512
MM = 512
NB = T // M
NBM = T // MM
WMAX = NBM + E - 1
XCOL = D + 128

NW = 32
TPW = T // NW
CS = 32
NCH = TPW // CS


def _gate_body(x_ref, wg_ref, tri_ref, tinc_ref, gs_ref, gs1_ref, eye_ref,
               gate_ref, rank_ref, xsc_ref, wl_ref, loss_ref,
               run_ref, psum_ref):
    b = pl.program_id(0)

    @pl.when(b == 0)
    def _():
        run_ref[...] = jnp.zeros_like(run_ref)
        psum_ref[...] = jnp.zeros_like(psum_ref)

    xb = x_ref[...]
    wg = wg_ref[...]
    logits = jnp.dot(xb, wg, preferred_element_type=jnp.float32)
    lane = lax.broadcasted_iota(jnp.int32, (M, EP), 1)
    valid = lane < E
    neg = jnp.full_like(logits, -jnp.inf)
    logit_m = jnp.where(valid, logits, neg)
    mx = jnp.max(logit_m, axis=-1, keepdims=True)
    ex = jnp.where(valid, jnp.exp(logit_m - mx), 0.0)
    den = jnp.sum(ex, axis=-1, keepdims=True)
    probs = ex / den
    gate = jnp.argmax(logit_m, axis=-1).astype(jnp.int32)
    selp = jnp.max(probs, axis=-1)

    onehot = jnp.where(lane == gate[:, None], 1.0, 0.0)
    cum_excl = jnp.dot(tri_ref[...], onehot, preferred_element_type=jnp.float32)
    local_rank = jnp.sum(cum_excl * onehot, axis=1)
    carry = jnp.sum(run_ref[...] * onehot, axis=1)
    rank = (local_rank + carry).astype(jnp.int32)

    gate_ref[...] = gate
    rank_ref[...] = rank
    run_ref[...] = run_ref[...] + jnp.sum(onehot, axis=0, keepdims=True)
    psum_ref[...] = psum_ref[...] + jnp.sum(probs, axis=0, keepdims=True)

    xsc_ref[:, :D] = xb * selp[:, None]
    lane2 = lax.broadcasted_iota(jnp.int32, (M, XCOL - D), 1)
    xsc_ref[:, D:] = jnp.where(lane2 == 0, selp[:, None], 0.0)

    @pl.when(b == NB - 1)
    def _():
        cnt = run_ref[...]
        psum = psum_ref[...]
        cum = jnp.dot(cnt, tinc_ref[...],
                      preferred_element_type=jnp.float32)
        exc = cum - cnt
        pv = lax.broadcasted_iota(jnp.int32, (1, EP), 1)
        bbf = (pv >> 3).astype(jnp.float32)
        eef = (pv & 7).astype(jnp.float32)
        offe = jnp.dot(exc, gs_ref[...], preferred_element_type=jnp.float32)
        offe1 = jnp.dot(exc, gs1_ref[...], preferred_element_type=jnp.float32)
        seg_s = jnp.maximum(offe, bbf * MM)
        seg_e = jnp.minimum(offe1, bbf * MM + MM)
        act = jnp.logical_and(seg_e > seg_s, pv < NBM * E)
        actf = jnp.where(act, 1.0, 0.0)
        slot = jnp.dot(actf, tinc_ref[...],
                       preferred_element_type=jnp.float32) - 1.0
        slot_m = jnp.where(act, slot, -1.0)
        slot_t = jnp.sum(eye_ref[...] * slot_m, axis=1, keepdims=True)
        wlane = lax.broadcasted_iota(jnp.int32, (EP, EP), 1).astype(jnp.float32)
        match = jnp.where(slot_t == wlane, 1.0, 0.0)
        vals = jnp.concatenate([bbf, eef, seg_s, seg_e], axis=0)
        out4 = jnp.dot(vals, match, preferred_element_type=jnp.float32)
        cw = jnp.sum(match, axis=0, keepdims=True)
        filled = cw > 0.0
        blk_l = jnp.where(filled, out4[0:1, :], float(NBM - 1))
        eid_l = jnp.where(filled, out4[1:2, :], 0.0)
        rs_l = jnp.where(filled, out4[2:3, :], 0.0)
        re_l = jnp.where(filled, out4[3:4, :], 0.0)
        wl_ref[0:1, :] = blk_l.astype(jnp.int32)
        wl_ref[1:2, :] = eid_l.astype(jnp.int32)
        wl_ref[2:3, :] = rs_l.astype(jnp.int32)
        wl_ref[3:4, :] = re_l.astype(jnp.int32)
        wl_ref[4:5, :] = exc.astype(jnp.int32)
        wl_ref[5:6, :] = cnt.astype(jnp.int32)
        wl_ref[6:8, :] = jnp.zeros((2, EP), jnp.int32)
        ftot = cnt / jnp.float32(T)
        loss = jnp.float32(E) * jnp.sum((psum / jnp.float32(T)) * ftot)
        loss_ref[...] = jnp.full((1, EP), loss, jnp.float32)


def _pos_chunks(gate_hbm, rank_hbm, wl_hbm, g_v, r_v, o_v, pos_v, base):
    pltpu.sync_copy(gate_hbm.at[pl.ds(base, TPW)], g_v)
    pltpu.sync_copy(rank_hbm.at[pl.ds(base, TPW)], r_v)
    pltpu.sync_copy(wl_hbm.at[4, pl.ds(0, 16)], o_v)
    for c in range(TPW // 16):
        g16 = g_v[pl.ds(c * 16, 16)]
        off16 = plsc.load_gather(o_v, [g16])
        j, k = divmod(c * 16, CS)
        pos_v[j, pl.ds(k, 16)] = off16 + r_v[pl.ds(c * 16, 16)]


def _dispatch_body(gate_hbm, rank_hbm, wl_hbm, xsc_hbm, xs_hbm,
                   g_v, r_v, o_v, pos_v, rows_v, sem):
    wid = lax.axis_index("s") * 2 + lax.axis_index("c")
    base = wid * TPW
    _pos_chunks(gate_hbm, rank_hbm, wl_hbm, g_v, r_v, o_v, pos_v, base)
    for j in range(NCH):
        pltpu.sync_copy(xsc_hbm.at[pl.ds(base + j * CS, CS)], rows_v)
        pltpu.async_copy(rows_v, xs_hbm.at[pos_v.at[j]], sem).wait()


def _combine_body(gate_hbm, rank_hbm, wl_hbm, ys_hbm, out_hbm,
                  g_v, r_v, o_v, pos_v, rows_v, sem):
    wid = lax.axis_index("s") * 2 + lax.axis_index("c")
    base = wid * TPW
    _pos_chunks(gate_hbm, rank_hbm, wl_hbm, g_v, r_v, o_v, pos_v, base)
    for j in range(NCH):
        pltpu.async_copy(ys_hbm.at[pos_v.at[j]], rows_v, sem).wait()
        pltpu.sync_copy(rows_v, out_hbm.at[pl.ds(base + j * CS, CS)])


@functools.cache
def _sc_kernels():
    mesh = plsc.VectorSubcoreMesh(core_axis_name="c", subcore_axis_name="s")
    params = pltpu.CompilerParams(needs_layout_passes=False)
    dispatch = pl.kernel(
        _dispatch_body, mesh=mesh, compiler_params=params,
        out_type=jax.ShapeDtypeStruct((T, XCOL), jnp.float32),
        scratch_types=[
            pltpu.VMEM((TPW,), jnp.int32),
            pltpu.VMEM((TPW,), jnp.int32),
            pltpu.VMEM((16,), jnp.int32),
            pltpu.VMEM((NCH, CS), jnp.int32),
            pltpu.VMEM((CS, XCOL), jnp.float32),
            pltpu.SemaphoreType.DMA,
        ],
    )
    combine = pl.kernel(
        _combine_body, mesh=mesh, compiler_params=params,
        out_type=jax.ShapeDtypeStruct((T, D), jnp.float32),
        scratch_types=[
            pltpu.VMEM((TPW,), jnp.int32),
            pltpu.VMEM((TPW,), jnp.int32),
            pltpu.VMEM((16,), jnp.int32),
            pltpu.VMEM((NCH, CS), jnp.int32),
            pltpu.VMEM((CS, D), jnp.float32),
            pltpu.SemaphoreType.DMA,
        ],
    )
    return dispatch, combine


def _moe_body(wl_s, xs_ref, w_ref, b_ref, ys_ref):
    w = pl.program_id(0)
    blk = wl_s[0, w]
    prev_blk = wl_s[0, jnp.maximum(w - 1, 0)]
    first = jnp.logical_or(w == 0, blk != prev_blk)
    xb = xs_ref[...]
    y = jnp.dot(xb[:, :D], w_ref[0], preferred_element_type=jnp.float32)
    y = y + xb[:, D:D + 1] * b_ref[0, 0, :][None, :]
    jg = blk * MM + lax.broadcasted_iota(jnp.int32, (MM, 1), 0)
    mask = jnp.logical_and(jg >= wl_s[2, w], jg < wl_s[3, w])
    contrib = jnp.where(mask, y, 0.0)
    ys_ref[...] = jnp.where(first, contrib, ys_ref[...] + contrib)


def kernel(x, attention_mask, W_gate, W_experts, b_experts):
    del attention_mask
    xf = x.reshape(T, D)
    wg_pad = jnp.zeros((D, EP), jnp.float32).at[:, :E].set(W_gate)
    tri = jnp.tril(jnp.ones((M, M), jnp.float32), -1)
    tinc = jnp.triu(jnp.ones((EP, EP), jnp.float32))
    jj = jnp.arange(EP, dtype=jnp.int32)[:, None]
    ppl = jnp.arange(EP, dtype=jnp.int32)[None, :]
    gs = ((jj == (ppl & 7)) & (ppl < NBM * E)).astype(jnp.float32)
    gs1 = ((jj == (ppl & 7) + 1) & (ppl < NBM * E)).astype(jnp.float32)
    eye = jnp.eye(EP, dtype=jnp.float32)

    gate, rank, xsc, wl, lossrow = pl.pallas_call(
        _gate_body,
        grid=(NB,),
        in_specs=[
            pl.BlockSpec((M, D), lambda b: (b, 0)),
            pl.BlockSpec((D, EP), lambda b: (0, 0)),
            pl.BlockSpec((M, M), lambda b: (0, 0)),
            pl.BlockSpec((EP, EP), lambda b: (0, 0)),
            pl.BlockSpec((EP, EP), lambda b: (0, 0)),
            pl.BlockSpec((EP, EP), lambda b: (0, 0)),
            pl.BlockSpec((EP, EP), lambda b: (0, 0)),
        ],
        out_specs=[
            pl.BlockSpec((M,), lambda b: (b,)),
            pl.BlockSpec((M,), lambda b: (b,)),
            pl.BlockSpec((M, XCOL), lambda b: (b, 0)),
            pl.BlockSpec((8, EP), lambda b: (0, 0)),
            pl.BlockSpec((1, EP), lambda b: (0, 0)),
        ],
        out_shape=[
            jax.ShapeDtypeStruct((T,), jnp.int32),
            jax.ShapeDtypeStruct((T,), jnp.int32),
            jax.ShapeDtypeStruct((T, XCOL), jnp.float32),
            jax.ShapeDtypeStruct((8, EP), jnp.int32),
            jax.ShapeDtypeStruct((1, EP), jnp.float32),
        ],
        scratch_shapes=[pltpu.VMEM((1, EP), jnp.float32),
                        pltpu.VMEM((1, EP), jnp.float32)],
    )(xf, wg_pad, tri, tinc, gs, gs1, eye)

    _dispatch, _combine = _sc_kernels()
    xs = _dispatch(gate, rank, wl, xsc)

    grid_spec = pltpu.PrefetchScalarGridSpec(
        num_scalar_prefetch=1,
        grid=(WMAX,),
        in_specs=[
            pl.BlockSpec((MM, XCOL), lambda w, s: (s[0, w], 0)),
            pl.BlockSpec((1, D, D), lambda w, s: (s[1, w], 0, 0)),
            pl.BlockSpec((1, 1, D), lambda w, s: (s[1, w], 0, 0)),
        ],
        out_specs=pl.BlockSpec((MM, D), lambda w, s: (s[0, w], 0)),
    )
    ys = pl.pallas_call(
        _moe_body,
        grid_spec=grid_spec,
        out_shape=jax.ShapeDtypeStruct((T, D), jnp.float32),
    )(wl, xs, W_experts, b_experts.reshape(E, 1, D))

    out = _combine(gate, rank, wl, ys)

    balance_loss = lossrow[0, 0]
    gate_load = wl[5, :E]
    return out.reshape(B, S, D), balance_loss, gate_load
